# baseline (speedup 1.0000x reference)
import functools
import jax
import jax.numpy as jnp
from jax import lax
from jax.experimental import pallas as pl
from jax.experimental.pallas import tpu as pltpu
from jax.experimental.pallas import tpu_sc as plsc

B = 1024
F = 4096
RB = 512
NB = B // RB
K_DROP = 103
N_SEL = 8
L = 16


def _sum_i16(x):
    r = x.shape[0]
    while r > 1:
        h = r // 2
        x = x[:h] + x[h:r]
        r = h
    return x


def _score_kernel_body(m_ref, idx_out_ref, dcol_ref, sc_ref, sqc_ref, sqr_ref):
    i = pl.program_id(0)
    j = pl.program_id(1)

    @pl.when((i == 0) & (j == 0))
    def _prep():
        m = m_ref[...]
        msq = m * m
        sqc_ref[...] = jnp.sum(msq, axis=1, keepdims=True)
        sqr_ref[...] = lax.dot_general(
            jnp.ones((1, F), jnp.float32), msq, (((1,), (1,)), ((), ())),
            preferred_element_type=jnp.float32)

    @pl.when(j >= i)
    def _matmul():
        mi = m_ref[pl.ds(i * RB, RB), :]
        mj = m_ref[pl.ds(j * RB, RB), :]
        g = lax.dot_general(mj, mi, (((1,), (1,)), ((), ())),
                            preferred_element_type=jnp.float32)
        sq_j = sqc_ref[pl.ds(j * RB, RB), :]
        sq_i = sqr_ref[:, pl.ds(i * RB, RB)]
        d2 = sq_j + sq_i - 2.0 * g
        d2 = jnp.maximum(d2, 0.0)
        safe = jnp.where(d2 > 0.0, d2, 1.0)
        d = jnp.where(d2 > 0.0, jnp.sqrt(safe), 0.0)
        dcol_ref[i, pl.ds(j * RB, RB), :] = d

        @pl.when(j > i)
        def _mirror():
            dcol_ref[j, pl.ds(i * RB, RB), :] = d.T

    @pl.when(j == NB - 1)
    def _score():
        dall = dcol_ref[i]
        bits = lax.bitcast_convert_type(dall, jnp.int32)
        rowsum = jnp.sum(dall, axis=0, keepdims=True)
        rowmin = jnp.min(dall, axis=0, keepdims=True)

        h16 = lax.shift_right_logical(bits, 16).astype(jnp.int16)
        lx16 = (bits ^ jnp.int32(0x8000)).astype(jnp.int16)
        one16 = jnp.int16(1)
        zero16 = jnp.int16(0)
        kdrop16 = jnp.int16(K_DROP)

        P = jnp.zeros((1, RB), jnp.int16)
        for b in range(14, -1, -1):
            cand = P | jnp.int16(1 << b)
            cnt = _sum_i16(jnp.where(h16 >= cand, one16, zero16))
            P = jnp.where(cnt >= kdrop16, cand, P)

        maskP = h16 == P
        c_hi = _sum_i16(jnp.where(h16 > P, one16, zero16))
        k_rem = kdrop16 - c_hi

        Lx = jnp.full((1, RB), jnp.int16(-0x8000))
        for b in range(15, -1, -1):
            if b == 15:
                cand = Lx & jnp.int16(0x7FFF)
            else:
                cand = Lx | jnp.int16(1 << b)
            hit = maskP & (lx16 >= cand)
            cnt = _sum_i16(jnp.where(hit, one16, zero16))
            Lx = jnp.where(cnt >= k_rem, cand, Lx)

        low_u = (Lx.astype(jnp.int32) ^ jnp.int32(0x8000)) & jnp.int32(0xFFFF)
        T = lax.shift_left(P.astype(jnp.int32), 16) | low_u
        gt = bits > T
        cnt_gt = jnp.sum(jnp.where(gt, 1.0, 0.0), axis=0, keepdims=True)
        sum_gt = jnp.sum(jnp.where(gt, dall, 0.0), axis=0, keepdims=True)
        kth = lax.bitcast_convert_type(T, jnp.float32)
        sumtop = sum_gt + (K_DROP - cnt_gt) * kth
        sc_ref[i] = rowsum - sumtop - rowmin

    @pl.when((i == NB - 1) & (j == NB - 1))
    def _select():
        s = sc_ref[...].reshape(NB, RB)
        iota = (lax.broadcasted_iota(jnp.int32, (NB, RB), 0) * RB
                + lax.broadcasted_iota(jnp.int32, (NB, RB), 1))
        lane = lax.broadcasted_iota(jnp.int32, (1, 128), 1)
        w0 = jnp.zeros((1, 128), jnp.int32)

        def pick(p, carry):
            s, w = carry
            m = jnp.min(s)
            elig = s == m
            idx = jnp.min(jnp.where(elig, iota, jnp.int32(2 ** 30)))
            onehot = iota == idx
            w = jnp.where(lane == p, idx, w)
            s = jnp.where(onehot, jnp.float32(jnp.inf), s)
            return s, w

        _, w = lax.fori_loop(0, N_SEL, pick, (s, w0))
        idx_out_ref[0] = w


def _sc_tail_body(idx_hbm, m_hbm, out_hbm, idx_v, rows_v, acc_v, sem):
    cid = lax.axis_index("c")
    sid = lax.axis_index("s")

    @pl.when((cid == 0) & (sid == 0))
    def _():
        pltpu.sync_copy(idx_hbm, idx_v)
        pltpu.async_copy(m_hbm.at[idx_v], rows_v, sem).wait()

        def col_body(c, _):
            acc = jnp.zeros((L,), jnp.float32)
            for r in range(N_SEL):
                acc = acc + rows_v[r, pl.ds(c * L, L)]
            acc_v[pl.ds(c * L, L)] = acc * (1.0 / N_SEL)
            return _

        lax.fori_loop(0, F // L, col_body, 0)
        pltpu.sync_copy(acc_v, out_hbm)


def kernel(matrix):
    idx3 = pl.pallas_call(
        _score_kernel_body,
        grid=(NB, NB),
        in_specs=[
            pl.BlockSpec((B, F), lambda i, j: (0, 0)),
        ],
        out_specs=pl.BlockSpec((1, 1, 128), lambda i, j: (0, 0, 0)),
        out_shape=jax.ShapeDtypeStruct((1, 1, 128), jnp.int32),
        scratch_shapes=[
            pltpu.VMEM((NB, B, RB), jnp.float32),
            pltpu.VMEM((NB, 1, RB), jnp.float32),
            pltpu.VMEM((B, 1), jnp.float32),
            pltpu.VMEM((1, B), jnp.float32),
        ],
    )(matrix)

    idx8 = idx3.reshape(128)[:N_SEL]

    mesh = plsc.VectorSubcoreMesh(core_axis_name="c", subcore_axis_name="s")
    sc_tail = functools.partial(
        pl.kernel, mesh=mesh,
        out_type=jax.ShapeDtypeStruct((F,), jnp.float32),
        scratch_types=[
            pltpu.VMEM((N_SEL,), jnp.int32),
            pltpu.VMEM((N_SEL, F), jnp.float32),
            pltpu.VMEM((F,), jnp.float32),
            pltpu.SemaphoreType.DMA,
        ],
    )(_sc_tail_body)

    return sc_tail(idx8, matrix)


# speedup vs baseline: 1.6277x; 1.6277x over previous
import jax
import jax.numpy as jnp
from jax import lax
from jax.experimental import pallas as pl
from jax.experimental.pallas import tpu as pltpu

B = 1024
F = 4096
RB = 512
NB = B // RB
K_DROP = 103
N_SEL = 8


def _sum_i16(x):
    r = x.shape[0]
    while r > 1:
        h = r // 2
        x = x[:h] + x[h:r]
        r = h
    return x


def _krum_body(m_ref, out_ref, dcol_ref, sc_ref, sqc_ref, sqr_ref):
    i = pl.program_id(0)
    j = pl.program_id(1)

    @pl.when((i == 0) & (j == 0))
    def _prep():
        m = m_ref[...]
        msq = m * m
        sqc_ref[...] = jnp.sum(msq, axis=1, keepdims=True)
        sqr_ref[...] = lax.dot_general(
            jnp.ones((1, F), jnp.float32), msq, (((1,), (1,)), ((), ())),
            preferred_element_type=jnp.float32)

    @pl.when(j >= i)
    def _matmul():
        mi = m_ref[pl.ds(i * RB, RB), :]
        mj = m_ref[pl.ds(j * RB, RB), :]
        g = lax.dot_general(mj, mi, (((1,), (1,)), ((), ())),
                            preferred_element_type=jnp.float32)
        sq_j = sqc_ref[pl.ds(j * RB, RB), :]
        sq_i = sqr_ref[:, pl.ds(i * RB, RB)]
        d2 = sq_j + sq_i - 2.0 * g
        d2 = jnp.maximum(d2, 0.0)
        safe = jnp.where(d2 > 0.0, d2, 1.0)
        d = jnp.where(d2 > 0.0, jnp.sqrt(safe), 0.0)
        dcol_ref[i, pl.ds(j * RB, RB), :] = d

        @pl.when(j > i)
        def _mirror():
            dcol_ref[j, pl.ds(i * RB, RB), :] = d.T

    @pl.when(j == NB - 1)
    def _score():
        dall = dcol_ref[i]
        bits = lax.bitcast_convert_type(dall, jnp.int32)
        rowsum = jnp.sum(dall, axis=0, keepdims=True)
        rowmin = jnp.min(dall, axis=0, keepdims=True)

        h16 = lax.shift_right_logical(bits, 16).astype(jnp.int16)
        lx16 = (bits ^ jnp.int32(0x8000)).astype(jnp.int16)
        one16 = jnp.int16(1)
        zero16 = jnp.int16(0)
        kdrop16 = jnp.int16(K_DROP)

        P = jnp.zeros((1, RB), jnp.int16)
        for b in range(14, -1, -1):
            cand = P | jnp.int16(1 << b)
            cnt = _sum_i16(jnp.where(h16 >= cand, one16, zero16))
            P = jnp.where(cnt >= kdrop16, cand, P)

        maskP = h16 == P
        c_hi = _sum_i16(jnp.where(h16 > P, one16, zero16))
        k_rem = kdrop16 - c_hi

        Lx = jnp.full((1, RB), jnp.int16(-0x8000))
        for b in range(15, -1, -1):
            if b == 15:
                cand = Lx & jnp.int16(0x7FFF)
            else:
                cand = Lx | jnp.int16(1 << b)
            hit = maskP & (lx16 >= cand)
            cnt = _sum_i16(jnp.where(hit, one16, zero16))
            Lx = jnp.where(cnt >= k_rem, cand, Lx)

        low_u = (Lx.astype(jnp.int32) ^ jnp.int32(0x8000)) & jnp.int32(0xFFFF)
        T = lax.shift_left(P.astype(jnp.int32), 16) | low_u
        gt = bits > T
        cnt_gt = jnp.sum(jnp.where(gt, 1.0, 0.0), axis=0, keepdims=True)
        sum_gt = jnp.sum(jnp.where(gt, dall, 0.0), axis=0, keepdims=True)
        kth = lax.bitcast_convert_type(T, jnp.float32)
        sumtop = sum_gt + (K_DROP - cnt_gt) * kth
        sc_ref[i] = rowsum - sumtop - rowmin

    @pl.when((i == NB - 1) & (j == NB - 1))
    def _select():
        s = sc_ref[...].reshape(NB, RB)
        iota = (lax.broadcasted_iota(jnp.int32, (NB, RB), 0) * RB
                + lax.broadcasted_iota(jnp.int32, (NB, RB), 1))
        w0 = jnp.zeros((NB, RB), jnp.float32)

        def pick(_, carry):
            s, w = carry
            m = jnp.min(s)
            elig = s == m
            idx = jnp.min(jnp.where(elig, iota, jnp.int32(2 ** 30)))
            onehot = iota == idx
            w = w + jnp.where(onehot, 1.0 / N_SEL, 0.0)
            s = jnp.where(onehot, jnp.float32(jnp.inf), s)
            return s, w

        _, w = lax.fori_loop(0, N_SEL, pick, (s, w0))
        acc = jnp.zeros((1, F), jnp.float32)
        for ib in range(NB):
            acc = acc + lax.dot_general(
                w[ib:ib + 1, :], m_ref[ib * RB:(ib + 1) * RB, :],
                (((1,), (0,)), ((), ())),
                preferred_element_type=jnp.float32)
        out_ref[0] = acc


def kernel(matrix):
    out3 = pl.pallas_call(
        _krum_body,
        grid=(NB, NB),
        in_specs=[
            pl.BlockSpec((B, F), lambda i, j: (0, 0)),
        ],
        out_specs=pl.BlockSpec((1, 1, F), lambda i, j: (0, 0, 0)),
        out_shape=jax.ShapeDtypeStruct((1, 1, F), jnp.float32),
        scratch_shapes=[
            pltpu.VMEM((NB, B, RB), jnp.float32),
            pltpu.VMEM((NB, 1, RB), jnp.float32),
            pltpu.VMEM((B, 1), jnp.float32),
            pltpu.VMEM((1, B), jnp.float32),
        ],
    )(matrix)

    return out3.reshape(F)
